# baseline (speedup 1.0000x reference)
import jax
import jax.numpy as jnp
from jax import lax
from jax.experimental import pallas as pl
from jax.experimental.pallas import tpu as pltpu
from jax.experimental.pallas import tpu_sc as plsc

B = 16384
V = 1000000
D = 64
Q = V // 4

NC = 2
NS = 16
NW = NC * NS
BPW = B // NW
CHUNK = 128
NCHUNK = BPW // CHUNK
NBUF = 6


def _sc_gather_body(uq_hbm, iq_hbm, gmf_t, mu_t, mi_t,
                    pm_o, qm_o, gu_o, gi_o,
                    uq_v, iq_v, bufs, gsem, ssem):
  wid = lax.axis_index("s") * NC + lax.axis_index("c")
  base = wid * BPW

  pltpu.sync_copy(uq_hbm.at[wid], uq_v)
  pltpu.sync_copy(iq_hbm.at[wid], iq_v)

  tasks = []
  for j in range(NCHUNK):
    tasks.append((mu_t, uq_v, j, pm_o))
    tasks.append((mi_t, iq_v, j, qm_o))
    tasks.append((gmf_t, uq_v, j, gu_o))
    tasks.append((gmf_t, iq_v, j, gi_o))

  NT = len(tasks)
  g = [None] * NT
  s = [None] * NT

  def fire(k):
    tbl, idx, j, _ = tasks[k]
    b = k % NBUF
    g[k] = pltpu.async_copy(tbl.at[idx.at[j]], bufs.at[b], gsem.at[b])

  def store(k):
    _, _, j, out = tasks[k]
    b = k % NBUF
    g[k].wait()
    s[k] = pltpu.async_copy(bufs.at[b],
                            out.at[pl.ds(base + j * CHUNK, CHUNK)],
                            ssem.at[b])

  for k in range(NT):
    if k >= NBUF:
      s[k - NBUF].wait()
    fire(k)
    if k >= NBUF - 1:
      store(k - (NBUF - 1))
  for k in range(NT - (NBUF - 1), NT):
    store(k)
  for k in range(NT - NBUF, NT):
    s[k].wait()


def _sc_gather(uq_r, iq_r, gmf_q, mu_q, mi_q):
  mesh = plsc.VectorSubcoreMesh(core_axis_name="c", subcore_axis_name="s")
  out = jax.ShapeDtypeStruct((B, 128), jnp.int32)
  run = pl.kernel(
      _sc_gather_body,
      out_type=[out, out, out, out],
      mesh=mesh,
      compiler_params=pltpu.CompilerParams(use_tc_tiling_on_sc=True),
      scratch_types=[
          pltpu.VMEM((NCHUNK, CHUNK), jnp.int32),
          pltpu.VMEM((NCHUNK, CHUNK), jnp.int32),
          pltpu.VMEM((NBUF, CHUNK, 128), jnp.int32),
          pltpu.SemaphoreType.DMA((NBUF,)),
          pltpu.SemaphoreType.DMA((NBUF,)),
      ],
  )
  return run(uq_r, iq_r, gmf_q, mu_q, mi_q)


def _extract_row(x_ref, hsel, ssel):
  xs = x_ref[...]
  ls = jnp.where(hsel == 0, xs[:, :D], xs[:, D:])
  lu = lax.bitcast_convert_type(ls, jnp.uint32)
  lo = lax.bitcast_convert_type(lu << 16, jnp.float32)
  hi = lax.bitcast_convert_type(lu & jnp.uint32(0xFFFF0000), jnp.float32)
  return jnp.where(ssel == 0, lo, hi)


def _tc_mlp_body(um_ref, im_ref, pm_ref, qm_ref, gu_ref, gi_ref,
                 w1a_ref, w1b_ref, b1_ref, w2t_ref, b2_ref, wog_ref, woh_ref,
                 out_ref):
  hi = jax.lax.Precision.HIGHEST
  f32 = jnp.float32
  um = um_ref[...]
  im = im_ref[...]
  uh, us = um % 2, (um % 4) // 2
  ih, i_s = im % 2, (im % 4) // 2

  pm = _extract_row(pm_ref, uh, us)
  qm = _extract_row(qm_ref, ih, i_s)
  gu = _extract_row(gu_ref, uh, us)
  gi = _extract_row(gi_ref, ih, i_s)

  h = jnp.dot(pm, w1a_ref[...], precision=hi, preferred_element_type=f32)
  h = h + jnp.dot(qm, w1b_ref[...], precision=hi, preferred_element_type=f32)
  h = h + b1_ref[...]
  h = jnp.where(h >= 0, h, 0.01 * h)
  h = jnp.dot(h, w2t_ref[...], precision=hi, preferred_element_type=f32) + b2_ref[...]
  h = jnp.where(h >= 0, h, 0.01 * h)
  gmf = gu * gi
  out = jnp.dot(gmf, wog_ref[...], precision=hi, preferred_element_type=f32)
  out = out + jnp.dot(h, woh_ref[...], precision=hi, preferred_element_type=f32)
  out_ref[...] = out


def _tc_mlp(um, im, pm, qm, gu, gi, w1a, w1b, b1, w2t, b2, wog, woh):
  R = 2048
  grid = B // R
  full = lambda shape: pl.BlockSpec(shape, lambda i: (0, 0))
  quad = pl.BlockSpec((R, 128), lambda i: (i, 0))
  return pl.pallas_call(
      _tc_mlp_body,
      grid=(grid,),
      in_specs=[
          pl.BlockSpec((R, 1), lambda i: (i, 0)),
          pl.BlockSpec((R, 1), lambda i: (i, 0)),
          quad, quad, quad, quad,
          full((D, 128)),
          full((D, 128)),
          full((1, 128)),
          full((128, D)),
          full((1, D)),
          full((D, 1)),
          full((D, 1)),
      ],
      out_specs=pl.BlockSpec((R, 1), lambda i: (i, 0)),
      out_shape=jax.ShapeDtypeStruct((B, 1), jnp.float32),
  )(um, im, pm, qm, gu, gi, w1a, w1b, b1, w2t, b2, wog, woh)


def _to_packed(t):
  u = lax.bitcast_convert_type(t, jnp.uint32)
  r = (u + jnp.uint32(0x7FFF) + ((u >> 16) & jnp.uint32(1))) >> 16
  rq = r.reshape(Q, 4, D)
  lo = rq[:, 0:2, :].reshape(Q, 128)
  hi = rq[:, 2:4, :].reshape(Q, 128)
  return lax.bitcast_convert_type(lo | (hi << 16), jnp.int32)


def kernel(user_id, item_id, gmf_item_emb, mlp_user_emb, mlp_item_emb,
           W1, b1, W2, b2, W_out):
  uid = user_id.astype(jnp.int32)
  iid = item_id.astype(jnp.int32)
  uq_r = (uid // 4).reshape(NW, NCHUNK, CHUNK)
  iq_r = (iid // 4).reshape(NW, NCHUNK, CHUNK)

  pm, qm, gu, gi = _sc_gather(uq_r, iq_r, _to_packed(gmf_item_emb),
                              _to_packed(mlp_user_emb),
                              _to_packed(mlp_item_emb))

  w1t = W1.T
  w1a = w1t[:D]
  w1b = w1t[D:]
  w2t = W2.T
  wot = W_out.T
  wog = wot[:D]
  woh = wot[D:]

  return _tc_mlp(uid.reshape(B, 1), iid.reshape(B, 1), pm, qm, gu, gi,
                 w1a, w1b, b1.reshape(1, 128), w2t, b2.reshape(1, D),
                 wog, woh)


# speedup vs baseline: 2.9836x; 2.9836x over previous
import jax
import jax.numpy as jnp
from jax import lax
from jax.experimental import pallas as pl
from jax.experimental.pallas import tpu as pltpu
from jax.experimental.pallas import tpu_sc as plsc

B = 16384
V = 1000000
D = 64
P = V // 2

NC = 2
NS = 16
NW = NC * NS
BPW = B // NW
CHUNK = 128
NCHUNK = BPW // CHUNK
NBUF = 6


def _sc_gather_body(uq_hbm, iq_hbm, gmf_t, mu_t, mi_t,
                    pm_o, qm_o, gu_o, gi_o,
                    uq_v, iq_v, bufs, gsem, ssem):
  wid = lax.axis_index("s") * NC + lax.axis_index("c")
  base = wid * BPW

  pltpu.sync_copy(uq_hbm.at[wid], uq_v)
  pltpu.sync_copy(iq_hbm.at[wid], iq_v)

  tasks = []
  for j in range(NCHUNK):
    tasks.append((mu_t, uq_v, j, pm_o))
    tasks.append((mi_t, iq_v, j, qm_o))
    tasks.append((gmf_t, uq_v, j, gu_o))
    tasks.append((gmf_t, iq_v, j, gi_o))

  NT = len(tasks)
  g = [None] * NT
  s = [None] * NT

  def fire(k):
    tbl, idx, j, _ = tasks[k]
    b = k % NBUF
    g[k] = pltpu.async_copy(tbl.at[idx.at[j]], bufs.at[b], gsem.at[b])

  def store(k):
    _, _, j, out = tasks[k]
    b = k % NBUF
    g[k].wait()
    s[k] = pltpu.async_copy(bufs.at[b],
                            out.at[pl.ds(base + j * CHUNK, CHUNK)],
                            ssem.at[b])

  for k in range(NT):
    if k >= NBUF:
      s[k - NBUF].wait()
    fire(k)
    if k >= NBUF - 1:
      store(k - (NBUF - 1))
  for k in range(NT - (NBUF - 1), NT):
    store(k)
  for k in range(NT - NBUF, NT):
    s[k].wait()


def _sc_gather(uq_r, iq_r, gmf_p, mu_p, mi_p):
  mesh = plsc.VectorSubcoreMesh(core_axis_name="c", subcore_axis_name="s")
  out = jax.ShapeDtypeStruct((B, 128), jnp.float32)
  run = pl.kernel(
      _sc_gather_body,
      out_type=[out, out, out, out],
      mesh=mesh,
      compiler_params=pltpu.CompilerParams(use_tc_tiling_on_sc=True),
      scratch_types=[
          pltpu.VMEM((NCHUNK, CHUNK), jnp.int32),
          pltpu.VMEM((NCHUNK, CHUNK), jnp.int32),
          pltpu.VMEM((NBUF, CHUNK, 128), jnp.float32),
          pltpu.SemaphoreType.DMA((NBUF,)),
          pltpu.SemaphoreType.DMA((NBUF,)),
      ],
  )
  return run(uq_r, iq_r, gmf_p, mu_p, mi_p)


def _extract_row(x_ref, hsel):
  xs = x_ref[...]
  return jnp.where(hsel == 0, xs[:, :D], xs[:, D:])


def _tc_mlp_body(um_ref, im_ref, pm_ref, qm_ref, gu_ref, gi_ref,
                 w1a_ref, w1b_ref, b1_ref, w2t_ref, b2_ref, wog_ref, woh_ref,
                 out_ref):
  hi = jax.lax.Precision.HIGHEST
  f32 = jnp.float32
  uh = um_ref[...] % 2
  ih = im_ref[...] % 2

  pm = _extract_row(pm_ref, uh)
  qm = _extract_row(qm_ref, ih)
  gu = _extract_row(gu_ref, uh)
  gi = _extract_row(gi_ref, ih)

  h = jnp.dot(pm, w1a_ref[...], precision=hi, preferred_element_type=f32)
  h = h + jnp.dot(qm, w1b_ref[...], precision=hi, preferred_element_type=f32)
  h = h + b1_ref[...]
  h = jnp.where(h >= 0, h, 0.01 * h)
  h = jnp.dot(h, w2t_ref[...], precision=hi, preferred_element_type=f32) + b2_ref[...]
  h = jnp.where(h >= 0, h, 0.01 * h)
  gmf = gu * gi
  out = jnp.dot(gmf, wog_ref[...], precision=hi, preferred_element_type=f32)
  out = out + jnp.dot(h, woh_ref[...], precision=hi, preferred_element_type=f32)
  out_ref[...] = out


def _tc_mlp(um, im, pm, qm, gu, gi, w1a, w1b, b1, w2t, b2, wog, woh):
  R = 2048
  grid = B // R
  full = lambda shape: pl.BlockSpec(shape, lambda i: (0, 0))
  pair = pl.BlockSpec((R, 128), lambda i: (i, 0))
  return pl.pallas_call(
      _tc_mlp_body,
      grid=(grid,),
      in_specs=[
          pl.BlockSpec((R, 1), lambda i: (i, 0)),
          pl.BlockSpec((R, 1), lambda i: (i, 0)),
          pair, pair, pair, pair,
          full((D, 128)),
          full((D, 128)),
          full((1, 128)),
          full((128, D)),
          full((1, D)),
          full((D, 1)),
          full((D, 1)),
      ],
      out_specs=pl.BlockSpec((R, 1), lambda i: (i, 0)),
      out_shape=jax.ShapeDtypeStruct((B, 1), jnp.float32),
  )(um, im, pm, qm, gu, gi, w1a, w1b, b1, w2t, b2, wog, woh)


def kernel(user_id, item_id, gmf_item_emb, mlp_user_emb, mlp_item_emb,
           W1, b1, W2, b2, W_out):
  uid = user_id.astype(jnp.int32)
  iid = item_id.astype(jnp.int32)
  uq_r = (uid // 2).reshape(NW, NCHUNK, CHUNK)
  iq_r = (iid // 2).reshape(NW, NCHUNK, CHUNK)

  pm, qm, gu, gi = _sc_gather(uq_r, iq_r,
                              gmf_item_emb.reshape(P, 128),
                              mlp_user_emb.reshape(P, 128),
                              mlp_item_emb.reshape(P, 128))

  w1t = W1.T
  w1a = w1t[:D]
  w1b = w1t[D:]
  w2t = W2.T
  wot = W_out.T
  wog = wot[:D]
  woh = wot[D:]

  return _tc_mlp(uid.reshape(B, 1), iid.reshape(B, 1), pm, qm, gu, gi,
                 w1a, w1b, b1.reshape(1, 128), w2t, b2.reshape(1, D),
                 wog, woh)
